# baseline (speedup 1.0000x reference)
import functools

import jax
import jax.numpy as jnp
from jax import lax
from jax.experimental import pallas as pl
from jax.experimental.pallas import tpu as pltpu
from jax.experimental.pallas import tpu_sc as plsc

NC = 2
NS = 16
NW = NC * NS
B = 80


def _round_up(a, b):
    return (a + b - 1) // b * b


@functools.partial(jax.jit, static_argnums=(3, 4))
def _sc_seg_sum(table, src3d, dst3d, with_cnt, fsplit):
    if fsplit:
        N, D = table.shape[1], table.shape[2]
    else:
        N, D = table.shape
    NBLK = src3d.shape[1]
    NP = _round_up(N + 1, NS * 128)
    RPT = NP // NS
    ZROWS = 128 if RPT % 128 == 0 else RPT
    assert RPT % ZROWS == 0
    SPT = RPT
    CL = NS * SPT

    mesh = plsc.VectorSubcoreMesh(core_axis_name="c", subcore_axis_name="s")

    out_type = [jax.ShapeDtypeStruct((NC, NP, D), jnp.float32)]
    if with_cnt:
        out_type.append(jax.ShapeDtypeStruct((NC, CL), jnp.float32))

    scratch = [
        pltpu.VMEM((NBLK, B), jnp.int32),
        pltpu.VMEM((NBLK, B), jnp.int32),
        pltpu.VMEM((B, D), jnp.float32),
        pltpu.VMEM((B, D), jnp.float32),
        pltpu.VMEM((ZROWS, D), jnp.float32),
        pltpu.VMEM((SPT,), jnp.float32),
        pltpu.SemaphoreType.DMA,
        pltpu.SemaphoreType.DMA,
        pltpu.SemaphoreType.DMA,
        pltpu.SemaphoreType.DMA,
        pltpu.SemaphoreType.DMA,
        pltpu.VMEM_SHARED((NP, D), jnp.float32),
        pltpu.VMEM_SHARED((CL,), jnp.float32),
    ]

    def body(table_h, src_h, dst_h, *rest):
        out_h = rest[0]
        rest = rest[1:]
        if with_cnt:
            cnt_h = rest[0]
            rest = rest[1:]
        (srcv, dstv, rows0, rows1, zbuf, zcnt,
         sem0, sem1, semf0, semf1, semc, acc, cacc) = rest
        c = lax.axis_index("c")
        s = lax.axis_index("s")
        tbl = table_h.at[c] if fsplit else table_h
        wid = s if fsplit else c * NS + s

        z16 = jnp.zeros((16,), jnp.float32)
        one16 = jnp.full((16,), 1.0, jnp.float32)
        dpc = D // 16

        def zb(i, _):
            zbuf[i // dpc, pl.ds((i % dpc) * 16, 16)] = z16
            return 0

        lax.fori_loop(0, ZROWS * dpc, zb, 0)
        for k in range(RPT // ZROWS):
            pltpu.sync_copy(zbuf, acc.at[pl.ds(s * RPT + k * ZROWS, ZROWS)])

        if with_cnt:
            def zc(i, _):
                zcnt[pl.ds(i * 16, 16)] = z16
                return 0

            lax.fori_loop(0, SPT // 16, zc, 0)
            pltpu.sync_copy(zcnt, cacc.at[pl.ds(s * SPT, SPT)])

        pltpu.sync_copy(src_h.at[wid], srcv)
        pltpu.sync_copy(dst_h.at[wid], dstv)

        if with_cnt:
            def oc(i, _):
                zcnt[pl.ds(i * 16, 16)] = one16
                return 0

            lax.fori_loop(0, B // 16, oc, 0)

        plsc.subcore_barrier()

        ones_b = zcnt.at[pl.ds(0, B)]
        assert NBLK % 2 == 0 and NBLK >= 4

        def scat(j, rows):
            pltpu.sync_copy(rows, acc.at[dstv.at[j]], add=True)
            if with_cnt:
                pltpu.sync_copy(ones_b, cacc.at[dstv.at[j]], add=True)

        pltpu.async_copy(tbl.at[srcv.at[0]], rows0, sem0)

        def step(k, _):
            j0 = 2 * k
            pltpu.async_copy(tbl.at[srcv.at[j0 + 1]], rows1, sem1)
            pltpu.make_async_copy(tbl.at[srcv.at[j0]], rows0, sem0).wait()
            scat(j0, rows0)
            pltpu.async_copy(tbl.at[srcv.at[j0 + 2]], rows0, sem0)
            pltpu.make_async_copy(tbl.at[srcv.at[j0 + 1]], rows1, sem1).wait()
            scat(j0 + 1, rows1)
            return 0

        lax.fori_loop(0, NBLK // 2 - 1, step, 0)
        jl = NBLK - 2
        pltpu.async_copy(tbl.at[srcv.at[jl + 1]], rows1, sem1)
        pltpu.make_async_copy(tbl.at[srcv.at[jl]], rows0, sem0).wait()
        scat(jl, rows0)
        pltpu.make_async_copy(tbl.at[srcv.at[jl + 1]], rows1, sem1).wait()
        scat(jl + 1, rows1)

        plsc.subcore_barrier()

        pltpu.sync_copy(acc.at[pl.ds(s * RPT, RPT)],
                        out_h.at[c].at[pl.ds(s * RPT, RPT)])
        if with_cnt:
            pltpu.sync_copy(cacc.at[pl.ds(s * SPT, SPT)],
                            cnt_h.at[c].at[pl.ds(s * SPT, SPT)])

    f = pl.kernel(
        body, out_type=out_type, mesh=mesh, scratch_types=scratch,
        compiler_params=pltpu.CompilerParams(use_tc_tiling_on_sc=False))
    return f(table, src3d, dst3d)


RB = 512


def _k1_body(x_ref, w1l_ref, w1r_ref, y1_ref, xr_ref):
    xb = x_ref[...]
    y1 = jnp.dot(xb, w1l_ref[...], preferred_element_type=jnp.float32, precision=lax.Precision.HIGHEST)
    h = y1.shape[1] // 2
    y1_ref[0] = y1[:, :h]
    y1_ref[1] = y1[:, h:]
    xr_ref[...] = jnp.dot(xb, w1r_ref[...], preferred_element_type=jnp.float32, precision=lax.Precision.HIGHEST)


def _k2_body(a0_ref, a1_ref, cnt_ref, xr_ref, b1_ref, w2l_ref, w2r_ref,
             wlin_ref, y2_ref, hr2_ref):
    agg = jnp.concatenate([a0_ref[0], a1_ref[0]], axis=1)
    cs = cnt_ref[0]
    inv = 1.0 / jnp.maximum(cs, 1.0)
    h = jnp.maximum(agg * inv[:, None] + b1_ref[...] + xr_ref[...], 0.0)
    w2lin = jnp.dot(w2l_ref[...], wlin_ref[...], preferred_element_type=jnp.float32, precision=lax.Precision.HIGHEST)
    w2rin = jnp.dot(w2r_ref[...], wlin_ref[...], preferred_element_type=jnp.float32, precision=lax.Precision.HIGHEST)
    y2_ref[...] = jnp.dot(h, w2lin, preferred_element_type=jnp.float32, precision=lax.Precision.HIGHEST)
    hr2_ref[...] = jnp.dot(h, w2rin, preferred_element_type=jnp.float32, precision=lax.Precision.HIGHEST)


def _k3_body(a0_ref, a1_ref, cnt_ref, hr2_ref, b2_ref, wlin_ref, blin_ref,
             out_ref):
    agg = a0_ref[0] + a1_ref[0]
    cs = cnt_ref[0]
    inv = 1.0 / jnp.maximum(cs, 1.0)
    bias = jnp.dot(b2_ref[...], wlin_ref[...], preferred_element_type=jnp.float32, precision=lax.Precision.HIGHEST)
    out_ref[...] = agg * inv[:, None] + hr2_ref[...] + bias + blin_ref[...]


def kernel(x, edge_index, W1l, b1l, W1r, W2l, b2l, W2r, Wlin, blin):
    N, D = x.shape
    H = W1l.shape[1]
    O = Wlin.shape[1]
    E = edge_index.shape[1]
    CL = _round_up(N, NS * 128)
    Hh = H // 2

    def _prep_idx(arr, parts, fill):
        pt = E // parts
        nb = -(-pt // B)
        nb += nb % 2
        if nb * B > pt:
            pad = jnp.full((parts, nb * B - pt), fill, jnp.int32)
            a = jnp.concatenate([arr.reshape(parts, pt), pad], axis=1)
        else:
            a = arr.reshape(parts, pt)
        return a.reshape(parts, nb, B)

    src_f = _prep_idx(edge_index[0], NS, 0)
    dst_f = _prep_idx(edge_index[1], NS, N)
    src_e = _prep_idx(edge_index[0], NW, 0)
    dst_e = _prep_idx(edge_index[1], NW, N)

    grid = ((N + RB - 1) // RB,)
    wspec = pl.BlockSpec((D, H), lambda i: (0, 0))

    y1s, xr = pl.pallas_call(
        _k1_body,
        grid=grid,
        in_specs=[pl.BlockSpec((RB, D), lambda i: (i, 0)), wspec, wspec],
        out_specs=[pl.BlockSpec((NC, RB, Hh), lambda i: (0, i, 0)),
                   pl.BlockSpec((RB, H), lambda i: (i, 0))],
        out_shape=[jax.ShapeDtypeStruct((NC, N, Hh), jnp.float32),
                   jax.ShapeDtypeStruct((N, H), jnp.float32)],
    )(x, W1l, W1r)

    agg1, cnt = _sc_seg_sum(y1s, src_f, dst_f, True, True)

    y2, hr2 = pl.pallas_call(
        _k2_body,
        grid=grid,
        in_specs=[
            pl.BlockSpec((1, RB, Hh), lambda i: (0, i, 0)),
            pl.BlockSpec((1, RB, Hh), lambda i: (1, i, 0)),
            pl.BlockSpec((NC, RB), lambda i: (0, i)),
            pl.BlockSpec((RB, H), lambda i: (i, 0)),
            pl.BlockSpec((1, H), lambda i: (0, 0)),
            pl.BlockSpec((H, H), lambda i: (0, 0)),
            pl.BlockSpec((H, H), lambda i: (0, 0)),
            pl.BlockSpec((H, O), lambda i: (0, 0)),
        ],
        out_specs=[pl.BlockSpec((RB, O), lambda i: (i, 0))] * 2,
        out_shape=[jax.ShapeDtypeStruct((N, O), jnp.float32)] * 2,
    )(agg1, agg1, cnt, xr, b1l.reshape(1, H), W2l, W2r, Wlin)

    agg2 = _sc_seg_sum(y2, src_e, dst_e, False, False)[0]

    out = pl.pallas_call(
        _k3_body,
        grid=grid,
        in_specs=[
            pl.BlockSpec((1, RB, O), lambda i: (0, i, 0)),
            pl.BlockSpec((1, RB, O), lambda i: (1, i, 0)),
            pl.BlockSpec((NC, RB), lambda i: (0, i)),
            pl.BlockSpec((RB, O), lambda i: (i, 0)),
            pl.BlockSpec((1, D), lambda i: (0, 0)),
            pl.BlockSpec((H, O), lambda i: (0, 0)),
            pl.BlockSpec((1, O), lambda i: (0, 0)),
        ],
        out_specs=pl.BlockSpec((RB, O), lambda i: (i, 0)),
        out_shape=jax.ShapeDtypeStruct((N, O), jnp.float32),
    )(agg2, agg2, cnt, hr2, b2l.reshape(1, D), Wlin, blin.reshape(1, O))

    return out


# speedup vs baseline: 1.1234x; 1.1234x over previous
import functools

import jax
import jax.numpy as jnp
from jax import lax
from jax.experimental import pallas as pl
from jax.experimental.pallas import tpu as pltpu
from jax.experimental.pallas import tpu_sc as plsc

NC = 2
NS = 16
NW = NC * NS
B = 80


def _round_up(a, b):
    return (a + b - 1) // b * b


@functools.partial(jax.jit, static_argnums=(3, 4))
def _sc_seg_sum(table, src3d, dst3d, with_cnt, fsplit):
    if fsplit:
        N, D = table.shape[1], table.shape[2]
    else:
        N, D = table.shape
    NBLK = src3d.shape[1]
    NP = _round_up(N + 1, NS * 128)
    RPT = NP // NS
    ZROWS = 128 if RPT % 128 == 0 else RPT
    assert RPT % ZROWS == 0
    SPT = RPT
    CL = NS * SPT

    mesh = plsc.VectorSubcoreMesh(core_axis_name="c", subcore_axis_name="s")

    out_type = [jax.ShapeDtypeStruct((NC, NP, D), jnp.float32)]
    if with_cnt:
        out_type.append(jax.ShapeDtypeStruct((NC, CL), jnp.float32))

    scratch = [
        pltpu.VMEM((NBLK, B), jnp.int32),
        pltpu.VMEM((NBLK, B), jnp.int32),
        pltpu.VMEM((B, D), jnp.float32),
        pltpu.VMEM((B, D), jnp.float32),
        pltpu.VMEM((ZROWS, D), jnp.float32),
        pltpu.VMEM((SPT,), jnp.float32),
        pltpu.SemaphoreType.DMA,
        pltpu.SemaphoreType.DMA,
        pltpu.SemaphoreType.DMA,
        pltpu.SemaphoreType.DMA,
        pltpu.SemaphoreType.DMA,
        pltpu.VMEM_SHARED((NP, D), jnp.float32),
        pltpu.VMEM_SHARED((CL,), jnp.float32),
    ]

    def body(table_h, src_h, dst_h, *rest):
        out_h = rest[0]
        rest = rest[1:]
        if with_cnt:
            cnt_h = rest[0]
            rest = rest[1:]
        (srcv, dstv, rows0, rows1, zbuf, zcnt,
         sem0, sem1, semf0, semf1, semc, acc, cacc) = rest
        c = lax.axis_index("c")
        s = lax.axis_index("s")
        tbl = table_h.at[c] if fsplit else table_h
        wid = s if fsplit else c * NS + s

        z16 = jnp.zeros((16,), jnp.float32)
        one16 = jnp.full((16,), 1.0, jnp.float32)
        dpc = D // 16

        def zb(i, _):
            zbuf[i // dpc, pl.ds((i % dpc) * 16, 16)] = z16
            return 0

        lax.fori_loop(0, ZROWS * dpc, zb, 0)
        for k in range(RPT // ZROWS):
            pltpu.sync_copy(zbuf, acc.at[pl.ds(s * RPT + k * ZROWS, ZROWS)])

        if with_cnt:
            def zc(i, _):
                zcnt[pl.ds(i * 16, 16)] = z16
                return 0

            lax.fori_loop(0, SPT // 16, zc, 0)
            pltpu.sync_copy(zcnt, cacc.at[pl.ds(s * SPT, SPT)])

        pltpu.sync_copy(src_h.at[wid], srcv)
        pltpu.sync_copy(dst_h.at[wid], dstv)

        if with_cnt:
            def oc(i, _):
                zcnt[pl.ds(i * 16, 16)] = one16
                return 0

            lax.fori_loop(0, B // 16, oc, 0)

        plsc.subcore_barrier()

        ones_b = zcnt.at[pl.ds(0, B)]

        def scat(j, rows):
            pltpu.sync_copy(rows, acc.at[dstv.at[j]], add=True)
            if with_cnt:
                pltpu.sync_copy(ones_b, cacc.at[dstv.at[j]], add=True)

        pltpu.async_copy(tbl.at[srcv.at[0]], rows0, sem0)

        def step(k, _):
            j0 = 2 * k
            pltpu.async_copy(tbl.at[srcv.at[j0 + 1]], rows1, sem1)
            pltpu.make_async_copy(tbl.at[srcv.at[j0]], rows0, sem0).wait()
            scat(j0, rows0)
            pltpu.async_copy(tbl.at[srcv.at[j0 + 2]], rows0, sem0)
            pltpu.make_async_copy(tbl.at[srcv.at[j0 + 1]], rows1, sem1).wait()
            scat(j0 + 1, rows1)
            return 0

        if NBLK % 2 == 1:
            lax.fori_loop(0, (NBLK - 1) // 2, step, 0)
            pltpu.make_async_copy(tbl.at[srcv.at[NBLK - 1]], rows0, sem0).wait()
            scat(NBLK - 1, rows0)
        else:
            lax.fori_loop(0, NBLK // 2 - 1, step, 0)
            jl = NBLK - 2
            pltpu.async_copy(tbl.at[srcv.at[jl + 1]], rows1, sem1)
            pltpu.make_async_copy(tbl.at[srcv.at[jl]], rows0, sem0).wait()
            scat(jl, rows0)
            pltpu.make_async_copy(tbl.at[srcv.at[jl + 1]], rows1, sem1).wait()
            scat(jl + 1, rows1)

        plsc.subcore_barrier()

        pltpu.sync_copy(acc.at[pl.ds(s * RPT, RPT)],
                        out_h.at[c].at[pl.ds(s * RPT, RPT)])
        if with_cnt:
            pltpu.sync_copy(cacc.at[pl.ds(s * SPT, SPT)],
                            cnt_h.at[c].at[pl.ds(s * SPT, SPT)])

    f = pl.kernel(
        body, out_type=out_type, mesh=mesh, scratch_types=scratch,
        compiler_params=pltpu.CompilerParams(use_tc_tiling_on_sc=False))
    return f(table, src3d, dst3d)


RB = 512


def _k1_body(x_ref, w1l_ref, w1r_ref, y1_ref, xr_ref):
    xb = x_ref[...]
    y1 = jnp.dot(xb, w1l_ref[...], preferred_element_type=jnp.float32, precision=lax.Precision.HIGHEST)
    h = y1.shape[1] // 2
    y1_ref[0] = y1[:, :h]
    y1_ref[1] = y1[:, h:]
    xr_ref[...] = jnp.dot(xb, w1r_ref[...], preferred_element_type=jnp.float32, precision=lax.Precision.HIGHEST)


def _k2_body(a0_ref, a1_ref, cnt_ref, xr_ref, b1_ref, w2l_ref, w2r_ref,
             wlin_ref, y2_ref, hr2_ref):
    agg = jnp.concatenate([a0_ref[0], a1_ref[0]], axis=1)
    cs = cnt_ref[0]
    inv = 1.0 / jnp.maximum(cs, 1.0)
    h = jnp.maximum(agg * inv[:, None] + b1_ref[...] + xr_ref[...], 0.0)
    w2lin = jnp.dot(w2l_ref[...], wlin_ref[...], preferred_element_type=jnp.float32, precision=lax.Precision.HIGHEST)
    w2rin = jnp.dot(w2r_ref[...], wlin_ref[...], preferred_element_type=jnp.float32, precision=lax.Precision.HIGHEST)
    y2_ref[...] = jnp.dot(h, w2lin, preferred_element_type=jnp.float32, precision=lax.Precision.HIGHEST)
    hr2_ref[...] = jnp.dot(h, w2rin, preferred_element_type=jnp.float32, precision=lax.Precision.HIGHEST)


def _k3_body(a0_ref, a1_ref, cnt_ref, hr2_ref, b2_ref, wlin_ref, blin_ref,
             out_ref):
    agg = a0_ref[0] + a1_ref[0]
    cs = cnt_ref[0]
    inv = 1.0 / jnp.maximum(cs, 1.0)
    bias = jnp.dot(b2_ref[...], wlin_ref[...], preferred_element_type=jnp.float32, precision=lax.Precision.HIGHEST)
    out_ref[...] = agg * inv[:, None] + hr2_ref[...] + bias + blin_ref[...]


def kernel(x, edge_index, W1l, b1l, W1r, W2l, b2l, W2r, Wlin, blin):
    N, D = x.shape
    H = W1l.shape[1]
    O = Wlin.shape[1]
    E = edge_index.shape[1]
    CL = _round_up(N, NS * 128)
    Hh = H // 2

    def _prep_idx(arr, parts, fill):
        pt = E // parts
        nb = -(-pt // B)
        if nb * B > pt:
            pad = jnp.full((parts, nb * B - pt), fill, jnp.int32)
            a = jnp.concatenate([arr.reshape(parts, pt), pad], axis=1)
        else:
            a = arr.reshape(parts, pt)
        return a.reshape(parts, nb, B)

    src_f = _prep_idx(edge_index[0], NS, 0)
    dst_f = _prep_idx(edge_index[1], NS, N)
    src_e = _prep_idx(edge_index[0], NW, 0)
    dst_e = _prep_idx(edge_index[1], NW, N)

    grid = ((N + RB - 1) // RB,)
    wspec = pl.BlockSpec((D, H), lambda i: (0, 0))

    y1s, xr = pl.pallas_call(
        _k1_body,
        grid=grid,
        in_specs=[pl.BlockSpec((RB, D), lambda i: (i, 0)), wspec, wspec],
        out_specs=[pl.BlockSpec((NC, RB, Hh), lambda i: (0, i, 0)),
                   pl.BlockSpec((RB, H), lambda i: (i, 0))],
        out_shape=[jax.ShapeDtypeStruct((NC, N, Hh), jnp.float32),
                   jax.ShapeDtypeStruct((N, H), jnp.float32)],
    )(x, W1l, W1r)

    agg1, cnt = _sc_seg_sum(y1s, src_f, dst_f, True, True)

    y2, hr2 = pl.pallas_call(
        _k2_body,
        grid=grid,
        in_specs=[
            pl.BlockSpec((1, RB, Hh), lambda i: (0, i, 0)),
            pl.BlockSpec((1, RB, Hh), lambda i: (1, i, 0)),
            pl.BlockSpec((NC, RB), lambda i: (0, i)),
            pl.BlockSpec((RB, H), lambda i: (i, 0)),
            pl.BlockSpec((1, H), lambda i: (0, 0)),
            pl.BlockSpec((H, H), lambda i: (0, 0)),
            pl.BlockSpec((H, H), lambda i: (0, 0)),
            pl.BlockSpec((H, O), lambda i: (0, 0)),
        ],
        out_specs=[pl.BlockSpec((RB, O), lambda i: (i, 0))] * 2,
        out_shape=[jax.ShapeDtypeStruct((N, O), jnp.float32)] * 2,
    )(agg1, agg1, cnt, xr, b1l.reshape(1, H), W2l, W2r, Wlin)

    agg2 = _sc_seg_sum(y2, src_e, dst_e, False, False)[0]

    out = pl.pallas_call(
        _k3_body,
        grid=grid,
        in_specs=[
            pl.BlockSpec((1, RB, O), lambda i: (0, i, 0)),
            pl.BlockSpec((1, RB, O), lambda i: (1, i, 0)),
            pl.BlockSpec((NC, RB), lambda i: (0, i)),
            pl.BlockSpec((RB, O), lambda i: (i, 0)),
            pl.BlockSpec((1, D), lambda i: (0, 0)),
            pl.BlockSpec((H, O), lambda i: (0, 0)),
            pl.BlockSpec((1, O), lambda i: (0, 0)),
        ],
        out_specs=pl.BlockSpec((RB, O), lambda i: (i, 0)),
        out_shape=jax.ShapeDtypeStruct((N, O), jnp.float32),
    )(agg2, agg2, cnt, hr2, b2l.reshape(1, D), Wlin, blin.reshape(1, O))

    return out


# speedup vs baseline: 1.1434x; 1.0178x over previous
import functools

import jax
import jax.numpy as jnp
from jax import lax
from jax.experimental import pallas as pl
from jax.experimental.pallas import tpu as pltpu
from jax.experimental.pallas import tpu_sc as plsc

NC = 2
NS = 16
NW = NC * NS
B = 80


def _round_up(a, b):
    return (a + b - 1) // b * b


@functools.partial(jax.jit, static_argnums=(3, 4))
def _sc_seg_sum(table, src3d, dst3d, with_cnt, fsplit):
    if fsplit:
        N, D = table.shape[1], table.shape[2]
    else:
        N, D = table.shape
    NBLK = src3d.shape[1]
    NP = _round_up(N + 1, NS * 128)
    RPT = NP // NS
    ZROWS = 128 if RPT % 128 == 0 else RPT
    assert RPT % ZROWS == 0
    SPT = RPT
    CL = NS * SPT

    mesh = plsc.VectorSubcoreMesh(core_axis_name="c", subcore_axis_name="s")

    out_type = [jax.ShapeDtypeStruct((NC, NP, D), jnp.float32)]
    if with_cnt:
        out_type.append(jax.ShapeDtypeStruct((NC, CL), jnp.float32))

    scratch = [
        pltpu.VMEM((NBLK, B), jnp.int32),
        pltpu.VMEM((NBLK, B), jnp.int32),
        pltpu.VMEM((B, D), jnp.float32),
        pltpu.VMEM((B, D), jnp.float32),
        pltpu.VMEM((ZROWS, D), jnp.float32),
        pltpu.VMEM((SPT,), jnp.float32),
        pltpu.SemaphoreType.DMA,
        pltpu.SemaphoreType.DMA,
        pltpu.SemaphoreType.DMA,
        pltpu.SemaphoreType.DMA,
        pltpu.SemaphoreType.DMA,
        pltpu.VMEM_SHARED((NP, D), jnp.float32),
        pltpu.VMEM_SHARED((CL,), jnp.float32),
    ]

    def body(table_h, src_h, dst_h, *rest):
        out_h = rest[0]
        rest = rest[1:]
        if with_cnt:
            cnt_h = rest[0]
            rest = rest[1:]
        (srcv, dstv, rows0, rows1, zbuf, zcnt,
         sem0, sem1, semf0, semf1, semc, acc, cacc) = rest
        c = lax.axis_index("c")
        s = lax.axis_index("s")
        tbl = table_h.at[c] if fsplit else table_h
        wid = s if fsplit else c * NS + s

        z16 = jnp.zeros((16,), jnp.float32)
        one16 = jnp.full((16,), 1.0, jnp.float32)
        dpc = D // 16

        def zb(i, _):
            zbuf[i // dpc, pl.ds((i % dpc) * 16, 16)] = z16
            return 0

        lax.fori_loop(0, ZROWS * dpc, zb, 0)
        for k in range(RPT // ZROWS):
            pltpu.sync_copy(zbuf, acc.at[pl.ds(s * RPT + k * ZROWS, ZROWS)])

        if with_cnt:
            def zc(i, _):
                zcnt[pl.ds(i * 16, 16)] = z16
                return 0

            lax.fori_loop(0, SPT // 16, zc, 0)
            pltpu.sync_copy(zcnt, cacc.at[pl.ds(s * SPT, SPT)])

        pltpu.sync_copy(src_h.at[wid], srcv)
        pltpu.sync_copy(dst_h.at[wid], dstv)

        if with_cnt:
            def oc(i, _):
                zcnt[pl.ds(i * 16, 16)] = one16
                return 0

            lax.fori_loop(0, B // 16, oc, 0)

        plsc.subcore_barrier()

        ones_b = zcnt.at[pl.ds(0, B)]

        def scat(j, rows):
            if with_cnt:
                pltpu.async_copy(ones_b, cacc.at[dstv.at[j]], semc, add=True)
            pltpu.sync_copy(rows, acc.at[dstv.at[j]], add=True)

        pltpu.async_copy(tbl.at[srcv.at[0]], rows0, sem0)

        def step(k, _):
            j0 = 2 * k
            pltpu.async_copy(tbl.at[srcv.at[j0 + 1]], rows1, sem1)
            pltpu.make_async_copy(tbl.at[srcv.at[j0]], rows0, sem0).wait()
            scat(j0, rows0)
            pltpu.async_copy(tbl.at[srcv.at[j0 + 2]], rows0, sem0)
            pltpu.make_async_copy(tbl.at[srcv.at[j0 + 1]], rows1, sem1).wait()
            scat(j0 + 1, rows1)
            return 0

        if NBLK % 2 == 1:
            lax.fori_loop(0, (NBLK - 1) // 2, step, 0)
            pltpu.make_async_copy(tbl.at[srcv.at[NBLK - 1]], rows0, sem0).wait()
            scat(NBLK - 1, rows0)
        else:
            lax.fori_loop(0, NBLK // 2 - 1, step, 0)
            jl = NBLK - 2
            pltpu.async_copy(tbl.at[srcv.at[jl + 1]], rows1, sem1)
            pltpu.make_async_copy(tbl.at[srcv.at[jl]], rows0, sem0).wait()
            scat(jl, rows0)
            pltpu.make_async_copy(tbl.at[srcv.at[jl + 1]], rows1, sem1).wait()
            scat(jl + 1, rows1)

        if with_cnt:
            def drain(i, _):
                pltpu.make_async_copy(ones_b, cacc.at[dstv.at[0]], semc).wait()
                return 0

            lax.fori_loop(0, NBLK, drain, 0)

        plsc.subcore_barrier()

        pltpu.sync_copy(acc.at[pl.ds(s * RPT, RPT)],
                        out_h.at[c].at[pl.ds(s * RPT, RPT)])
        if with_cnt:
            pltpu.sync_copy(cacc.at[pl.ds(s * SPT, SPT)],
                            cnt_h.at[c].at[pl.ds(s * SPT, SPT)])

    f = pl.kernel(
        body, out_type=out_type, mesh=mesh, scratch_types=scratch,
        compiler_params=pltpu.CompilerParams(use_tc_tiling_on_sc=False))
    return f(table, src3d, dst3d)


RB = 512


def _k1_body(x_ref, w1l_ref, w1r_ref, y1_ref, xr_ref):
    xb = x_ref[...]
    y1 = jnp.dot(xb, w1l_ref[...], preferred_element_type=jnp.float32, precision=lax.Precision.HIGHEST)
    h = y1.shape[1] // 2
    y1_ref[0] = y1[:, :h]
    y1_ref[1] = y1[:, h:]
    xr_ref[...] = jnp.dot(xb, w1r_ref[...], preferred_element_type=jnp.float32, precision=lax.Precision.HIGHEST)


def _k2_body(a0_ref, a1_ref, cnt_ref, xr_ref, b1_ref, w2l_ref, w2r_ref,
             wlin_ref, y2_ref, hr2_ref):
    agg = jnp.concatenate([a0_ref[0], a1_ref[0]], axis=1)
    cs = cnt_ref[0]
    inv = 1.0 / jnp.maximum(cs, 1.0)
    h = jnp.maximum(agg * inv[:, None] + b1_ref[...] + xr_ref[...], 0.0)
    w2lin = jnp.dot(w2l_ref[...], wlin_ref[...], preferred_element_type=jnp.float32, precision=lax.Precision.HIGHEST)
    w2rin = jnp.dot(w2r_ref[...], wlin_ref[...], preferred_element_type=jnp.float32, precision=lax.Precision.HIGHEST)
    y2_ref[...] = jnp.dot(h, w2lin, preferred_element_type=jnp.float32, precision=lax.Precision.HIGHEST)
    hr2_ref[...] = jnp.dot(h, w2rin, preferred_element_type=jnp.float32, precision=lax.Precision.HIGHEST)


def _k3_body(a0_ref, a1_ref, cnt_ref, hr2_ref, b2_ref, wlin_ref, blin_ref,
             out_ref):
    agg = a0_ref[0] + a1_ref[0]
    cs = cnt_ref[0]
    inv = 1.0 / jnp.maximum(cs, 1.0)
    bias = jnp.dot(b2_ref[...], wlin_ref[...], preferred_element_type=jnp.float32, precision=lax.Precision.HIGHEST)
    out_ref[...] = agg * inv[:, None] + hr2_ref[...] + bias + blin_ref[...]


def kernel(x, edge_index, W1l, b1l, W1r, W2l, b2l, W2r, Wlin, blin):
    N, D = x.shape
    H = W1l.shape[1]
    O = Wlin.shape[1]
    E = edge_index.shape[1]
    CL = _round_up(N, NS * 128)
    Hh = H // 2

    def _prep_idx(arr, parts, fill):
        pt = E // parts
        nb = -(-pt // B)
        if nb * B > pt:
            pad = jnp.full((parts, nb * B - pt), fill, jnp.int32)
            a = jnp.concatenate([arr.reshape(parts, pt), pad], axis=1)
        else:
            a = arr.reshape(parts, pt)
        return a.reshape(parts, nb, B)

    src_f = _prep_idx(edge_index[0], NS, 0)
    dst_f = _prep_idx(edge_index[1], NS, N)
    src_e = _prep_idx(edge_index[0], NW, 0)
    dst_e = _prep_idx(edge_index[1], NW, N)

    grid = ((N + RB - 1) // RB,)
    wspec = pl.BlockSpec((D, H), lambda i: (0, 0))

    y1s, xr = pl.pallas_call(
        _k1_body,
        grid=grid,
        in_specs=[pl.BlockSpec((RB, D), lambda i: (i, 0)), wspec, wspec],
        out_specs=[pl.BlockSpec((NC, RB, Hh), lambda i: (0, i, 0)),
                   pl.BlockSpec((RB, H), lambda i: (i, 0))],
        out_shape=[jax.ShapeDtypeStruct((NC, N, Hh), jnp.float32),
                   jax.ShapeDtypeStruct((N, H), jnp.float32)],
    )(x, W1l, W1r)

    agg1, cnt = _sc_seg_sum(y1s, src_f, dst_f, True, True)

    y2, hr2 = pl.pallas_call(
        _k2_body,
        grid=grid,
        in_specs=[
            pl.BlockSpec((1, RB, Hh), lambda i: (0, i, 0)),
            pl.BlockSpec((1, RB, Hh), lambda i: (1, i, 0)),
            pl.BlockSpec((NC, RB), lambda i: (0, i)),
            pl.BlockSpec((RB, H), lambda i: (i, 0)),
            pl.BlockSpec((1, H), lambda i: (0, 0)),
            pl.BlockSpec((H, H), lambda i: (0, 0)),
            pl.BlockSpec((H, H), lambda i: (0, 0)),
            pl.BlockSpec((H, O), lambda i: (0, 0)),
        ],
        out_specs=[pl.BlockSpec((RB, O), lambda i: (i, 0))] * 2,
        out_shape=[jax.ShapeDtypeStruct((N, O), jnp.float32)] * 2,
    )(agg1, agg1, cnt, xr, b1l.reshape(1, H), W2l, W2r, Wlin)

    agg2 = _sc_seg_sum(y2, src_e, dst_e, False, False)[0]

    out = pl.pallas_call(
        _k3_body,
        grid=grid,
        in_specs=[
            pl.BlockSpec((1, RB, O), lambda i: (0, i, 0)),
            pl.BlockSpec((1, RB, O), lambda i: (1, i, 0)),
            pl.BlockSpec((NC, RB), lambda i: (0, i)),
            pl.BlockSpec((RB, O), lambda i: (i, 0)),
            pl.BlockSpec((1, D), lambda i: (0, 0)),
            pl.BlockSpec((H, O), lambda i: (0, 0)),
            pl.BlockSpec((1, O), lambda i: (0, 0)),
        ],
        out_specs=pl.BlockSpec((RB, O), lambda i: (i, 0)),
        out_shape=jax.ShapeDtypeStruct((N, O), jnp.float32),
    )(agg2, agg2, cnt, hr2, b2l.reshape(1, D), Wlin, blin.reshape(1, O))

    return out
